# Initial kernel scaffold; baseline (speedup 1.0000x reference)
import jax
import jax.numpy as jnp
from jax.experimental import pallas as pl


def kernel(inputs):
    raise NotImplementedError("write your pallas kernel here")


# speedup vs baseline: 244.8910x; 244.8910x over previous
import functools

import jax
import jax.numpy as jnp
from jax.experimental import pallas as pl
from jax.experimental.pallas import tpu as pltpu

S = 256
F = 2048
NEAR = 0.1
FAR = 100.0
EPS = 1e-8
FBLK = 256
NCOEF = 16


def _coef_body(v_ref, c_ref):
    X0 = v_ref[0]; X1 = v_ref[1]; X2 = v_ref[2]
    Y0 = v_ref[3]; Y1 = v_ref[4]; Y2 = v_ref[5]
    Z0 = v_ref[6]; Z1 = v_ref[7]; Z2 = v_ref[8]
    a0 = Y1 - Y2; b0 = X2 - X1; c0 = X1 * Y2 - X2 * Y1
    a1 = Y2 - Y0; b1 = X0 - X2; c1 = X2 * Y0 - X0 * Y2
    a2 = Y0 - Y1; b2 = X1 - X0; c2 = X0 * Y1 - X1 * Y0
    det = c0 + c1 + c2
    sgn = jnp.where(det >= 0.0, 1.0, -1.0)
    adet = jnp.abs(det)
    valid = adet > EPS
    iZ0 = 1.0 / jnp.where(jnp.abs(Z0) > EPS, Z0, 1.0)
    iZ1 = 1.0 / jnp.where(jnp.abs(Z1) > EPS, Z1, 1.0)
    iZ2 = 1.0 / jnp.where(jnp.abs(Z2) > EPS, Z2, 1.0)
    ga = a0 * iZ0 + a1 * iZ1 + a2 * iZ2
    gb = b0 * iZ0 + b1 * iZ1 + b2 * iZ2
    gc = c0 * iZ0 + c1 * iZ1 + c2 * iZ2
    big = jnp.float32(1e30)
    c_ref[0] = a0 * sgn
    c_ref[1] = b0 * sgn
    c_ref[2] = c0 * sgn
    c_ref[3] = a1 * sgn
    c_ref[4] = b1 * sgn
    c_ref[5] = c1 * sgn
    c_ref[6] = a2 * sgn
    c_ref[7] = b2 * sgn
    c_ref[8] = c2 * sgn
    c_ref[9] = ga * sgn
    c_ref[10] = gb * sgn
    c_ref[11] = gc * sgn
    c_ref[12] = jnp.where(valid, 1.0 / adet, 0.0)
    c_ref[13] = jnp.where(valid, adet * (1.0 / FAR), big)
    c_ref[14] = jnp.where(valid, adet * (1.0 / NEAR), -big)
    c_ref[15] = jnp.zeros_like(det)


def _raster_body(c_ref, out_ref, den_ref):
    fb = pl.program_id(1)

    @pl.when(fb == 0)
    def _init():
        den_ref[...] = jnp.full((S, S), 1.0 / FAR, jnp.float32)
        out_ref[0] = jnp.full((S, S), -1, jnp.int32)

    Xi = jax.lax.broadcasted_iota(jnp.int32, (S, S), 1).astype(jnp.float32)
    Yi = jax.lax.broadcasted_iota(jnp.int32, (S, S), 0).astype(jnp.float32)
    X = (2.0 * Xi + (1.0 - S)) / S
    Y = (2.0 * Yi + (1.0 - S)) / S

    def body(j, _):
        a0 = c_ref[0, 0, 0, j]; b0 = c_ref[0, 0, 1, j]; c0 = c_ref[0, 0, 2, j]
        a1 = c_ref[0, 0, 3, j]; b1 = c_ref[0, 0, 4, j]; c1 = c_ref[0, 0, 5, j]
        a2 = c_ref[0, 0, 6, j]; b2 = c_ref[0, 0, 7, j]; c2 = c_ref[0, 0, 8, j]
        ga = c_ref[0, 0, 9, j]; gb = c_ref[0, 0, 10, j]; gc = c_ref[0, 0, 11, j]
        radet = c_ref[0, 0, 12, j]; lo = c_ref[0, 0, 13, j]; hi = c_ref[0, 0, 14, j]
        w0 = a0 * X + (b0 * Y + c0)
        w1 = a1 * X + (b1 * Y + c1)
        w2 = a2 * X + (b2 * Y + c2)
        g = ga * X + (gb * Y + gc)
        den = g * radet
        m = (w0 >= 0.0) & (w1 >= 0.0) & (w2 >= 0.0) & (g > lo) & (g < hi)
        dold = den_ref[...]
        upd = m & (den > dold)
        den_ref[...] = jnp.where(upd, den, dold)
        fid = fb * FBLK + j
        out_ref[0] = jnp.where(upd, fid, out_ref[0])
        return 0

    jax.lax.fori_loop(0, FBLK, body, 0, unroll=False)


def kernel(inputs):
    B = inputs.shape[0]
    v = jnp.transpose(inputs, (3, 2, 0, 1)).reshape(9, B, F)
    coef = pl.pallas_call(
        _coef_body,
        out_shape=jax.ShapeDtypeStruct((NCOEF, B, F), jnp.float32),
    )(v)
    coef = jnp.transpose(coef.reshape(NCOEF, B, F // FBLK, FBLK), (1, 2, 0, 3))
    out = pl.pallas_call(
        _raster_body,
        grid=(B, F // FBLK),
        in_specs=[
            pl.BlockSpec((1, 1, NCOEF, FBLK), lambda b, fb: (b, fb, 0, 0),
                         memory_space=pltpu.SMEM),
        ],
        out_specs=pl.BlockSpec((1, S, S), lambda b, fb: (b, 0, 0)),
        out_shape=jax.ShapeDtypeStruct((B, S, S), jnp.int32),
        scratch_shapes=[pltpu.VMEM((S, S), jnp.float32)],
    )(coef)
    return out
